# Initial kernel scaffold; baseline (speedup 1.0000x reference)
import jax
import jax.numpy as jnp
from jax.experimental import pallas as pl


def kernel(h, z, pos, mask, e_feat, z_emb, pair_w0, pair_b0, pair_w1, pair_b1, pair_w2, pair_b2, geom_w0, geom_b0, geom_w1, geom_b1, geom_w2, geom_b2, out_w0, out_b0, out_w1, out_b1, absorber_index):
    raise NotImplementedError("write your pallas kernel here")


# speedup vs baseline: 1.2347x; 1.2347x over previous
import functools
import math

import jax
import jax.numpy as jnp
from jax import lax
from jax.experimental import pallas as pl

_B = 16
_N = 128
_ATOM = 128
_RBF = 32
_GHID = 128
_SDIM = 32
_ODIM = 64
_CUT = 6.0
_PMAX = 256
_ZEMB = 32
_EDIM = 32
_NE = 64
_PHID = 64
_INF_BITS = 0x7F800000

_HI = lax.Precision.HIGHEST


def _dotg(a, b):
  return lax.dot_general(a, b, (((0,), (0,)), ((), ())), precision=_HI)


def _dot(a, b):
  return jnp.dot(a, b, precision=_HI)


def _silu(x):
  return x * (1.0 / (1.0 + jnp.exp(-x)))


def _coscut(r):
  return 0.5 * (jnp.cos(math.pi * r / _CUT) + 1.0) * (r < _CUT).astype(r.dtype)


def _fiota(shape, dim):
  return lax.broadcasted_iota(jnp.int32, shape, dim).astype(jnp.float32)


def _body(aux_ref, h_ref, zemb_ref, ef_ref,
          pw0_ref, pb0_ref, pw1_ref, pb1_ref, pw2_ref, pb2_ref,
          gw0_ref, gb0_ref, gw1_ref, gb1_ref, gw2_ref, gb2_ref,
          ow0_ref, ob0_ref, ow1_ref, ob1_ref,
          out_ref):
  f32 = jnp.float32
  aux = aux_ref[0]
  hblk = h_ref[0]

  lane = _fiota((_N, _N), 1)
  sub = _fiota((_N, _N), 0)
  eye = (sub == lane).astype(f32)
  sub_col = _fiota((_N, 1), 0)

  a_col = jnp.sum(aux * (lane == 5.0).astype(f32), axis=1, keepdims=True)
  m_col = jnp.sum(aux * (lane == 3.0).astype(f32), axis=1, keepdims=True)

  cmask3 = (lane < 3.0).astype(f32)
  p3 = aux * cmask3
  a_oh = (sub_col == a_col).astype(f32)
  pos0row = _dotg(a_oh, p3)

  d0 = p3 - pos0row
  r2 = jnp.sum(d0 * d0, axis=1, keepdims=True)
  r_col = jnp.sqrt(r2)
  r_row = _dotg(r_col, eye)

  valid_col = ((m_col > 0.5) & (sub_col != a_col) &
               (r_col <= _CUT)).astype(f32)
  valid_row = _dotg(valid_col, eye)

  d2 = jnp.zeros((_N, _N), f32)
  for d in range(3):
    x_col = jnp.sum(aux * (lane == float(d)).astype(f32), axis=1,
                    keepdims=True)
    x_row = _dotg(x_col, eye)
    dd = x_col - x_row
    d2 = d2 + dd * dd
  dmat = jnp.sqrt(d2)

  score = r_col + r_row + 0.5 * dmat
  tri = (sub < lane).astype(f32)
  pvb = (valid_col > 0.5) & (valid_row > 0.5) & (sub < lane)
  si = jnp.where(pvb, lax.bitcast_convert_type(score, jnp.int32), _INF_BITS)

  def bis(_, lohi):
    lo, hi = lohi
    mid = lo + (hi - lo) // 2
    cnt = jnp.sum((si <= mid).astype(jnp.int32))
    take = cnt >= _PMAX
    return (jnp.where(take, lo, mid + 1), jnp.where(take, mid, hi))

  _, thr = lax.fori_loop(0, 31, bis, (jnp.int32(0), jnp.int32(_INF_BITS)))

  s01 = (pvb & (si <= thr)).astype(f32)
  rank = _dot(s01, tri)
  rc_col = jnp.sum(s01, axis=1, keepdims=True)
  off_col = _dotg(tri, rc_col)
  end_col = off_col + rc_col
  off_row = _dotg(off_col, eye)
  end_row = _dotg(end_col, eye)

  s_io = _fiota((_PMAX, _N), 0)
  rowsel = ((off_row <= s_io) & (s_io < end_row)).astype(f32)
  s_colv = _fiota((_PMAX, 1), 0)
  within = s_colv - _dot(rowsel, off_col)
  rank_rows = _dot(rowsel, rank)
  sel_rows = _dot(rowsel, s01)
  ksel = ((rank_rows == within) & (sel_rows > 0.5)).astype(f32)
  used = jnp.sum(rowsel, axis=1, keepdims=True)

  gj = _dot(rowsel, aux)
  gk = _dot(ksel, aux)
  hj = _dot(rowsel, hblk)
  hk = _dot(ksel, hblk)

  lane256 = _fiota((_PMAX, _N), 1)
  c3 = (lane256 < 3.0).astype(f32)
  vj = (gj - pos0row) * c3
  vk = (gk - pos0row) * c3
  vjk = (gk - gj) * c3
  r0j = jnp.sqrt(jnp.sum(vj * vj, axis=1, keepdims=True))
  r0k = jnp.sqrt(jnp.sum(vk * vk, axis=1, keepdims=True))
  rjk = jnp.sqrt(jnp.sum(vjk * vjk, axis=1, keepdims=True))
  uj = vj / jnp.maximum(r0j, 1e-8)
  uk = vk / jnp.maximum(r0k, 1e-8)
  cosang = jnp.clip(jnp.sum(uj * uk, axis=1, keepdims=True), -1.0, 1.0)

  offs = _fiota((_PMAX, _RBF), 1) * (_CUT / (_RBF - 1))
  coeff = -0.5 / (_CUT / (_RBF - 1)) ** 2

  def rbf(r):
    rr = jnp.minimum(r, _CUT)
    return jnp.exp(coeff * (rr - offs) ** 2)

  f0j = rbf(r0j)
  f0k = rbf(r0k)
  fjk = rbf(rjk)

  gw0 = gw0_ref[...]
  pre = (_dot(hj, gw0[0:128]) + _dot(hk, gw0[128:256])
         + _dot(f0j, gw0[256:288]) + _dot(f0k, gw0[288:320])
         + _dot(fjk, gw0[320:352]) + cosang * gw0[352:353]
         + gb0_ref[...])
  g1 = _silu(pre)
  g2 = _silu(_dot(g1, gw1_ref[...]) + gb1_ref[...])
  ggeom = _dot(g2, gw2_ref[...]) + gb2_ref[...]

  pw0 = pw0_ref[...]
  zemb = zemb_ref[...]
  zwj = _dot(zemb, pw0[0:32])
  zwk = _dot(zemb, pw0[32:64])
  efb = _dot(ef_ref[...], pw0[64:96]) + pb0_ref[...]

  zj = jnp.sum(gj * (lane256 == 4.0).astype(f32), axis=1, keepdims=True)
  zk = jnp.sum(gk * (lane256 == 4.0).astype(f32), axis=1, keepdims=True)
  zjoh = (lane256 == zj).astype(f32)
  zkoh = (lane256 == zk).astype(f32)
  ej = _dot(zjoh, zwj)
  ek = _dot(zkoh, zwk)

  pre0 = ej[:, None, :] + ek[:, None, :] + efb[None, :, :]
  x1 = _silu(pre0).reshape(_PMAX * _NE, _PHID)
  x2 = _silu(_dot(x1, pw1_ref[...]) + pb1_ref[...])
  ge = (_dot(x2, pw2_ref[...]) + pb2_ref[...]).reshape(_PMAX, _NE, _SDIM)

  cw = _coscut(r0j) * _coscut(r0k) * _coscut(rjk)
  w = cw * used
  amat = ggeom * w
  agg = jnp.sum(ge * amat[:, None, :], axis=0)
  norm = jnp.maximum(jnp.sum(w), 1e-8)
  agg = agg / norm

  o1 = _silu(_dot(agg, ow0_ref[...]) + ob0_ref[...])
  out = _dot(o1, ow1_ref[...]) + ob1_ref[...]
  out_ref[...] = out[None]


@jax.jit
def kernel(h, z, pos, mask, e_feat, z_emb,
           pair_w0, pair_b0, pair_w1, pair_b1, pair_w2, pair_b2,
           geom_w0, geom_b0, geom_w1, geom_b1, geom_w2, geom_b2,
           out_w0, out_b0, out_w1, out_b1, absorber_index):
  f32 = jnp.float32
  b = h.shape[0]
  aux = jnp.zeros((b, _N, _N), f32)
  aux = aux.at[:, :, 0:3].set(pos.astype(f32))
  aux = aux.at[:, :, 3].set(mask.astype(f32))
  aux = aux.at[:, :, 4].set(z.astype(f32))
  aux = aux.at[:, :, 5].set(jnp.asarray(absorber_index, f32))

  zemb_pad = jnp.zeros((_N, _ZEMB), f32).at[:z_emb.shape[0]].set(z_emb)

  def fullspec(x):
    r = x.ndim
    return pl.BlockSpec(x.shape, lambda i, _r=r: (0,) * _r)

  ins = (aux, h, zemb_pad, e_feat,
         pair_w0, pair_b0.reshape(1, -1), pair_w1, pair_b1.reshape(1, -1),
         pair_w2, pair_b2.reshape(1, -1),
         geom_w0, geom_b0.reshape(1, -1), geom_w1, geom_b1.reshape(1, -1),
         geom_w2, geom_b2.reshape(1, -1),
         out_w0, out_b0.reshape(1, -1), out_w1, out_b1.reshape(1, -1))

  specs = [pl.BlockSpec((1, _N, _N), lambda i: (i, 0, 0)),
           pl.BlockSpec((1, _N, _ATOM), lambda i: (i, 0, 0))]
  specs += [fullspec(x) for x in ins[2:]]

  return pl.pallas_call(
      _body,
      grid=(b,),
      in_specs=specs,
      out_specs=pl.BlockSpec((1, _NE, _ODIM), lambda i: (i, 0, 0)),
      out_shape=jax.ShapeDtypeStruct((b, _NE, _ODIM), f32),
  )(*ins)


# speedup vs baseline: 2.9613x; 2.3983x over previous
import functools
import math

import jax
import jax.numpy as jnp
from jax import lax
from jax.experimental import pallas as pl

_B = 16
_N = 128
_ATOM = 128
_RBF = 32
_GHID = 128
_SDIM = 32
_ODIM = 64
_CUT = 6.0
_PMAX = 256
_ZEMB = 32
_EDIM = 32
_NE = 64
_PHID = 64
_INF_BITS = 0x7F800000

_HI = lax.Precision.HIGHEST


def _dotg(a, b):
  return lax.dot_general(a, b, (((0,), (0,)), ((), ())), precision=_HI)


def _dot(a, b):
  return jnp.dot(a, b, precision=_HI)


def _dotb(a, b):
  return lax.dot_general(a.astype(jnp.bfloat16), b.astype(jnp.bfloat16),
                         (((1,), (0,)), ((), ())),
                         preferred_element_type=jnp.float32)


def _silu(x):
  return x * (1.0 / (1.0 + jnp.exp(-x)))


def _coscut(r):
  return 0.5 * (jnp.cos(math.pi * r / _CUT) + 1.0) * (r < _CUT).astype(r.dtype)


def _fiota(shape, dim):
  return lax.broadcasted_iota(jnp.int32, shape, dim).astype(jnp.float32)


def _body(aux_ref, h_ref, zemb_ref, ef_ref,
          pw0_ref, pb0_ref, pw1_ref, pb1_ref, pw2_ref, pb2_ref,
          gw0_ref, gb0_ref, gw1_ref, gb1_ref, gw2_ref, gb2_ref,
          ow0_ref, ob0_ref, ow1_ref, ob1_ref,
          out_ref):
  f32 = jnp.float32
  aux = aux_ref[0]
  hblk = h_ref[0]

  lane = _fiota((_N, _N), 1)
  sub = _fiota((_N, _N), 0)
  eye = (sub == lane).astype(f32)
  sub_col = _fiota((_N, 1), 0)

  a_col = jnp.sum(aux * (lane == 5.0).astype(f32), axis=1, keepdims=True)
  m_col = jnp.sum(aux * (lane == 3.0).astype(f32), axis=1, keepdims=True)

  cmask3 = (lane < 3.0).astype(f32)
  p3 = aux * cmask3
  a_oh = (sub_col == a_col).astype(f32)
  pos0row = _dotg(a_oh, p3)

  d0 = p3 - pos0row
  r2 = jnp.sum(d0 * d0, axis=1, keepdims=True)
  r_col = jnp.sqrt(r2)
  r_row = _dotg(r_col, eye)

  valid_col = ((m_col > 0.5) & (sub_col != a_col) &
               (r_col <= _CUT)).astype(f32)
  valid_row = _dotg(valid_col, eye)

  d2 = jnp.zeros((_N, _N), f32)
  for d in range(3):
    x_col = jnp.sum(aux * (lane == float(d)).astype(f32), axis=1,
                    keepdims=True)
    x_row = _dotg(x_col, eye)
    dd = x_col - x_row
    d2 = d2 + dd * dd
  dmat = jnp.sqrt(d2)

  score = r_col + r_row + 0.5 * dmat
  tri = (sub < lane).astype(f32)
  pvb = (valid_col > 0.5) & (valid_row > 0.5) & (sub < lane)
  si = jnp.where(pvb, lax.bitcast_convert_type(score, jnp.int32), _INF_BITS)

  def bis(_, lohi):
    lo, hi = lohi
    mid = lo + (hi - lo) // 2
    cnt = jnp.sum((si <= mid).astype(jnp.int32))
    take = cnt >= _PMAX
    return (jnp.where(take, lo, mid + 1), jnp.where(take, mid, hi))

  _, thr = lax.fori_loop(0, 31, bis, (jnp.int32(0), jnp.int32(_INF_BITS)))

  s01 = (pvb & (si <= thr)).astype(f32)
  rank = _dot(s01, tri)
  rc_col = jnp.sum(s01, axis=1, keepdims=True)
  off_col = _dotg(tri, rc_col)
  end_col = off_col + rc_col
  off_row = _dotg(off_col, eye)
  end_row = _dotg(end_col, eye)

  s_io = _fiota((_PMAX, _N), 0)
  rowsel = ((off_row <= s_io) & (s_io < end_row)).astype(f32)
  s_colv = _fiota((_PMAX, 1), 0)
  within = s_colv - _dot(rowsel, off_col)
  rank_rows = _dot(rowsel, rank)
  sel_rows = _dot(rowsel, s01)
  ksel = ((rank_rows == within) & (sel_rows > 0.5)).astype(f32)
  used = jnp.sum(rowsel, axis=1, keepdims=True)

  gj = _dot(rowsel, aux)
  gk = _dot(ksel, aux)
  hj = _dot(rowsel, hblk)
  hk = _dot(ksel, hblk)

  lane256 = _fiota((_PMAX, _N), 1)
  c3 = (lane256 < 3.0).astype(f32)
  vj = (gj - pos0row) * c3
  vk = (gk - pos0row) * c3
  vjk = (gk - gj) * c3
  r0j = jnp.sqrt(jnp.sum(vj * vj, axis=1, keepdims=True))
  r0k = jnp.sqrt(jnp.sum(vk * vk, axis=1, keepdims=True))
  rjk = jnp.sqrt(jnp.sum(vjk * vjk, axis=1, keepdims=True))
  uj = vj / jnp.maximum(r0j, 1e-8)
  uk = vk / jnp.maximum(r0k, 1e-8)
  cosang = jnp.clip(jnp.sum(uj * uk, axis=1, keepdims=True), -1.0, 1.0)

  offs = _fiota((_PMAX, _RBF), 1) * (_CUT / (_RBF - 1))
  coeff = -0.5 / (_CUT / (_RBF - 1)) ** 2

  def rbf(r):
    rr = jnp.minimum(r, _CUT)
    return jnp.exp(coeff * (rr - offs) ** 2)

  f0j = rbf(r0j)
  f0k = rbf(r0k)
  fjk = rbf(rjk)

  gw0 = gw0_ref[...]
  pre = (_dotb(hj, gw0[0:128]) + _dotb(hk, gw0[128:256])
         + _dotb(f0j, gw0[256:288]) + _dotb(f0k, gw0[288:320])
         + _dotb(fjk, gw0[320:352]) + cosang * gw0[352:353]
         + gb0_ref[...])
  g1 = _silu(pre)
  g2 = _silu(_dotb(g1, gw1_ref[...]) + gb1_ref[...])
  ggeom = _dotb(g2, gw2_ref[...]) + gb2_ref[...]

  pw0 = pw0_ref[...]
  zemb = zemb_ref[...]
  zwj = _dot(zemb, pw0[0:32])
  zwk = _dot(zemb, pw0[32:64])
  efb = _dot(ef_ref[...], pw0[64:96]) + pb0_ref[...]

  zj = jnp.sum(gj * (lane256 == 4.0).astype(f32), axis=1, keepdims=True)
  zk = jnp.sum(gk * (lane256 == 4.0).astype(f32), axis=1, keepdims=True)
  zjoh = (lane256 == zj).astype(f32)
  zkoh = (lane256 == zk).astype(f32)
  ej = _dot(zjoh, zwj)
  ek = _dot(zkoh, zwk)

  pre0 = ej[:, None, :] + ek[:, None, :] + efb[None, :, :]
  x1 = _silu(pre0).reshape(_PMAX * _NE, _PHID)
  x2 = _silu(_dotb(x1, pw1_ref[...]) + pb1_ref[...])
  ge = (_dotb(x2, pw2_ref[...]) + pb2_ref[...]).reshape(_PMAX, _NE, _SDIM)

  cw = _coscut(r0j) * _coscut(r0k) * _coscut(rjk)
  w = cw * used
  amat = ggeom * w
  agg = jnp.sum(ge * amat[:, None, :], axis=0)
  norm = jnp.maximum(jnp.sum(w), 1e-8)
  agg = agg / norm

  o1 = _silu(_dot(agg, ow0_ref[...]) + ob0_ref[...])
  out = _dot(o1, ow1_ref[...]) + ob1_ref[...]
  out_ref[...] = out[None]


@jax.jit
def kernel(h, z, pos, mask, e_feat, z_emb,
           pair_w0, pair_b0, pair_w1, pair_b1, pair_w2, pair_b2,
           geom_w0, geom_b0, geom_w1, geom_b1, geom_w2, geom_b2,
           out_w0, out_b0, out_w1, out_b1, absorber_index):
  f32 = jnp.float32
  b = h.shape[0]
  aux = jnp.zeros((b, _N, _N), f32)
  aux = aux.at[:, :, 0:3].set(pos.astype(f32))
  aux = aux.at[:, :, 3].set(mask.astype(f32))
  aux = aux.at[:, :, 4].set(z.astype(f32))
  aux = aux.at[:, :, 5].set(jnp.asarray(absorber_index, f32))

  zemb_pad = jnp.zeros((_N, _ZEMB), f32).at[:z_emb.shape[0]].set(z_emb)

  def fullspec(x):
    r = x.ndim
    return pl.BlockSpec(x.shape, lambda i, _r=r: (0,) * _r)

  ins = (aux, h, zemb_pad, e_feat,
         pair_w0, pair_b0.reshape(1, -1), pair_w1, pair_b1.reshape(1, -1),
         pair_w2, pair_b2.reshape(1, -1),
         geom_w0, geom_b0.reshape(1, -1), geom_w1, geom_b1.reshape(1, -1),
         geom_w2, geom_b2.reshape(1, -1),
         out_w0, out_b0.reshape(1, -1), out_w1, out_b1.reshape(1, -1))

  specs = [pl.BlockSpec((1, _N, _N), lambda i: (i, 0, 0)),
           pl.BlockSpec((1, _N, _ATOM), lambda i: (i, 0, 0))]
  specs += [fullspec(x) for x in ins[2:]]

  return pl.pallas_call(
      _body,
      grid=(b,),
      in_specs=specs,
      out_specs=pl.BlockSpec((1, _NE, _ODIM), lambda i: (i, 0, 0)),
      out_shape=jax.ShapeDtypeStruct((b, _NE, _ODIM), f32),
  )(*ins)


# speedup vs baseline: 3.9941x; 1.3488x over previous
import functools
import math

import jax
import jax.numpy as jnp
from jax import lax
from jax.experimental import pallas as pl

_B = 16
_N = 128
_ATOM = 128
_RBF = 32
_GHID = 128
_SDIM = 32
_ODIM = 64
_CUT = 6.0
_PMAX = 256
_ZEMB = 32
_EDIM = 32
_NE = 64
_PHID = 64
_INF_BITS = 0x7F800000

_HI = lax.Precision.HIGHEST


def _dotg(a, b):
  return lax.dot_general(a, b, (((0,), (0,)), ((), ())), precision=_HI)


def _dot(a, b):
  return jnp.dot(a, b, precision=_HI)


def _dotb(a, b):
  return lax.dot_general(a.astype(jnp.bfloat16), b.astype(jnp.bfloat16),
                         (((1,), (0,)), ((), ())),
                         preferred_element_type=jnp.float32)


def _silu(x):
  return x * (1.0 / (1.0 + jnp.exp(-x)))


def _coscut(r):
  return 0.5 * (jnp.cos(math.pi * r / _CUT) + 1.0) * (r < _CUT).astype(r.dtype)


def _fiota(shape, dim):
  return lax.broadcasted_iota(jnp.int32, shape, dim).astype(jnp.float32)


def _body(aux_ref, h_ref, zemb_ref, ef_ref,
          pw0_ref, pw0e2_ref, pb0b_ref, pw1b_ref, pb1b_ref, pw2b_ref,
          pb2b_ref,
          gw0_ref, gb0_ref, gw1_ref, gb1_ref, gw2_ref, gb2_ref,
          ow0_ref, ob0_ref, ow1_ref, ob1_ref,
          out_ref):
  f32 = jnp.float32
  aux = aux_ref[0]
  hblk = h_ref[0]

  lane = _fiota((_N, _N), 1)
  sub = _fiota((_N, _N), 0)
  eye = (sub == lane).astype(f32)
  sub_col = _fiota((_N, 1), 0)

  a_col = jnp.sum(aux * (lane == 5.0).astype(f32), axis=1, keepdims=True)
  m_col = jnp.sum(aux * (lane == 3.0).astype(f32), axis=1, keepdims=True)

  cmask3 = (lane < 3.0).astype(f32)
  p3 = aux * cmask3
  a_oh = (sub_col == a_col).astype(f32)
  pos0row = _dotg(a_oh, p3)

  d0 = p3 - pos0row
  r2 = jnp.sum(d0 * d0, axis=1, keepdims=True)
  r_col = jnp.sqrt(r2)
  r_row = _dotg(r_col, eye)

  valid_col = ((m_col > 0.5) & (sub_col != a_col) &
               (r_col <= _CUT)).astype(f32)
  valid_row = _dotg(valid_col, eye)

  d2 = jnp.zeros((_N, _N), f32)
  for d in range(3):
    x_col = jnp.sum(aux * (lane == float(d)).astype(f32), axis=1,
                    keepdims=True)
    x_row = _dotg(x_col, eye)
    dd = x_col - x_row
    d2 = d2 + dd * dd
  dmat = jnp.sqrt(d2)

  score = r_col + r_row + 0.5 * dmat
  tri = (sub < lane).astype(f32)
  pvb = (valid_col > 0.5) & (valid_row > 0.5) & (sub < lane)
  si = jnp.where(pvb, lax.bitcast_convert_type(score, jnp.int32), _INF_BITS)

  def bis(_, lohi):
    lo, hi = lohi
    d = (hi - lo) // 4
    m1 = lo + d
    m2 = lo + 2 * d
    m3 = lo + 3 * d
    c1 = jnp.sum((si <= m1).astype(jnp.int32)) >= _PMAX
    c2 = jnp.sum((si <= m2).astype(jnp.int32)) >= _PMAX
    c3 = jnp.sum((si <= m3).astype(jnp.int32)) >= _PMAX
    hi_n = jnp.where(c1, m1, jnp.where(c2, m2, jnp.where(c3, m3, hi)))
    lo_n = jnp.where(c1, lo,
                     jnp.where(c2, m1 + 1, jnp.where(c3, m2 + 1, m3 + 1)))
    return (lo_n, hi_n)

  _, thr = lax.fori_loop(0, 16, bis, (jnp.int32(0), jnp.int32(_INF_BITS)))

  s01 = (pvb & (si <= thr)).astype(f32)
  rank = _dot(s01, tri)
  rc_col = jnp.sum(s01, axis=1, keepdims=True)
  off_col = _dotg(tri, rc_col)
  end_col = off_col + rc_col
  off_row = _dotg(off_col, eye)
  end_row = _dotg(end_col, eye)

  s_io = _fiota((_PMAX, _N), 0)
  rowsel = ((off_row <= s_io) & (s_io < end_row)).astype(f32)
  s_colv = _fiota((_PMAX, 1), 0)
  within = s_colv - _dot(rowsel, off_col)
  rank_rows = _dot(rowsel, rank)
  sel_rows = _dot(rowsel, s01)
  ksel = ((rank_rows == within) & (sel_rows > 0.5)).astype(f32)
  used = jnp.sum(rowsel, axis=1, keepdims=True)

  gj = _dot(rowsel, aux)
  gk = _dot(ksel, aux)
  hj = _dot(rowsel, hblk)
  hk = _dot(ksel, hblk)

  lane256 = _fiota((_PMAX, _N), 1)
  c3 = (lane256 < 3.0).astype(f32)
  vj = (gj - pos0row) * c3
  vk = (gk - pos0row) * c3
  vjk = (gk - gj) * c3
  r0j = jnp.sqrt(jnp.sum(vj * vj, axis=1, keepdims=True))
  r0k = jnp.sqrt(jnp.sum(vk * vk, axis=1, keepdims=True))
  rjk = jnp.sqrt(jnp.sum(vjk * vjk, axis=1, keepdims=True))
  uj = vj / jnp.maximum(r0j, 1e-8)
  uk = vk / jnp.maximum(r0k, 1e-8)
  cosang = jnp.clip(jnp.sum(uj * uk, axis=1, keepdims=True), -1.0, 1.0)

  offs = _fiota((_PMAX, _RBF), 1) * (_CUT / (_RBF - 1))
  coeff = -0.5 / (_CUT / (_RBF - 1)) ** 2

  def rbf(r):
    rr = jnp.minimum(r, _CUT)
    return jnp.exp(coeff * (rr - offs) ** 2)

  f0j = rbf(r0j)
  f0k = rbf(r0k)
  fjk = rbf(rjk)

  gw0 = gw0_ref[...]
  pre = (_dotb(hj, gw0[0:128]) + _dotb(hk, gw0[128:256])
         + _dotb(f0j, gw0[256:288]) + _dotb(f0k, gw0[288:320])
         + _dotb(fjk, gw0[320:352]) + cosang * gw0[352:353]
         + gb0_ref[...])
  g1 = _silu(pre)
  g2 = _silu(_dotb(g1, gw1_ref[...]) + gb1_ref[...])
  ggeom = _dotb(g2, gw2_ref[...]) + gb2_ref[...]

  pw0 = pw0_ref[...]
  zemb = zemb_ref[...]
  zwj = _dot(zemb, pw0[0:32])
  zwk = _dot(zemb, pw0[32:64])
  efbp = _dot(ef_ref[...], pw0e2_ref[...]) + pb0b_ref[...]

  zj = jnp.sum(gj * (lane256 == 4.0).astype(f32), axis=1, keepdims=True)
  zk = jnp.sum(gk * (lane256 == 4.0).astype(f32), axis=1, keepdims=True)
  zjoh = (lane256 == zj).astype(f32)
  zkoh = (lane256 == zk).astype(f32)
  ej = _dot(zjoh, zwj)
  ek = _dot(zkoh, zwk)

  ejk = ej + ek
  ejk2 = jnp.concatenate([ejk, ejk], axis=1)
  pre0 = (ejk2[:, None, :] + efbp[None, :, :]).reshape(
      _PMAX * _NE // 2, 2 * _PHID)
  x1 = _silu(pre0)
  x2 = _silu(_dotb(x1, pw1b_ref[...]) + pb1b_ref[...])
  gep = _dotb(x2, pw2b_ref[...]) + pb2b_ref[...]

  cw = _coscut(r0j) * _coscut(r0k) * _coscut(rjk)
  w = cw * used
  amat = ggeom * w
  am2 = jnp.concatenate([amat, amat], axis=1)
  ge3 = gep.reshape(_PMAX, _NE // 2, 2 * _SDIM)
  aggp = jnp.sum(ge3 * am2[:, None, :], axis=0)
  n64 = lax.broadcasted_iota(jnp.int32, (_NE, _NE // 2), 0)
  nn2 = lax.broadcasted_iota(jnp.int32, (_NE, _NE // 2), 1) * 2
  r_e = (n64 == nn2).astype(f32)
  r_o = (n64 == nn2 + 1).astype(f32)
  agg = _dot(r_e, aggp[:, :_SDIM]) + _dot(r_o, aggp[:, _SDIM:])
  norm = jnp.maximum(jnp.sum(w), 1e-8)
  agg = agg / norm

  o1 = _silu(_dot(agg, ow0_ref[...]) + ob0_ref[...])
  out = _dot(o1, ow1_ref[...]) + ob1_ref[...]
  out_ref[...] = out[None]


@jax.jit
def kernel(h, z, pos, mask, e_feat, z_emb,
           pair_w0, pair_b0, pair_w1, pair_b1, pair_w2, pair_b2,
           geom_w0, geom_b0, geom_w1, geom_b1, geom_w2, geom_b2,
           out_w0, out_b0, out_w1, out_b1, absorber_index):
  f32 = jnp.float32
  b = h.shape[0]
  aux = jnp.zeros((b, _N, _N), f32)
  aux = aux.at[:, :, 0:3].set(pos.astype(f32))
  aux = aux.at[:, :, 3].set(mask.astype(f32))
  aux = aux.at[:, :, 4].set(z.astype(f32))
  aux = aux.at[:, :, 5].set(jnp.asarray(absorber_index, f32))

  zemb_pad = jnp.zeros((_N, _ZEMB), f32).at[:z_emb.shape[0]].set(z_emb)

  ef_pack = jnp.concatenate([e_feat[0::2], e_feat[1::2]], axis=1)
  w0e = pair_w0[2 * _ZEMB:]
  pw0e2 = (jnp.zeros((2 * _EDIM, 2 * _PHID), f32)
           .at[:_EDIM, :_PHID].set(w0e).at[_EDIM:, _PHID:].set(w0e))
  pb0b = jnp.concatenate([pair_b0, pair_b0]).reshape(1, -1)

  pw1b = (jnp.zeros((2 * _PHID, 2 * _PHID), f32)
          .at[:_PHID, :_PHID].set(pair_w1).at[_PHID:, _PHID:].set(pair_w1))
  pb1b = jnp.concatenate([pair_b1, pair_b1]).reshape(1, -1)
  pw2b = (jnp.zeros((2 * _PHID, 2 * _SDIM), f32)
          .at[:_PHID, :_SDIM].set(pair_w2).at[_PHID:, _SDIM:].set(pair_w2))
  pb2b = jnp.concatenate([pair_b2, pair_b2]).reshape(1, -1)

  def fullspec(x):
    r = x.ndim
    return pl.BlockSpec(x.shape, lambda i, _r=r: (0,) * _r)

  ins = (aux, h, zemb_pad, ef_pack,
         pair_w0, pw0e2, pb0b, pw1b, pb1b, pw2b, pb2b,
         geom_w0, geom_b0.reshape(1, -1), geom_w1, geom_b1.reshape(1, -1),
         geom_w2, geom_b2.reshape(1, -1),
         out_w0, out_b0.reshape(1, -1), out_w1, out_b1.reshape(1, -1))

  specs = [pl.BlockSpec((1, _N, _N), lambda i: (i, 0, 0)),
           pl.BlockSpec((1, _N, _ATOM), lambda i: (i, 0, 0))]
  specs += [fullspec(x) for x in ins[2:]]

  return pl.pallas_call(
      _body,
      grid=(b,),
      in_specs=specs,
      out_specs=pl.BlockSpec((1, _NE, _ODIM), lambda i: (i, 0, 0)),
      out_shape=jax.ShapeDtypeStruct((b, _NE, _ODIM), f32),
  )(*ins)


# speedup vs baseline: 4.3659x; 1.0931x over previous
import functools
import math

import jax
import jax.numpy as jnp
from jax import lax
from jax.experimental import pallas as pl

_B = 16
_N = 128
_ATOM = 128
_RBF = 32
_GHID = 128
_SDIM = 32
_ODIM = 64
_CUT = 6.0
_PMAX = 256
_ZEMB = 32
_EDIM = 32
_NE = 64
_PHID = 64
_INF_BITS = 0x7F800000

_HI = lax.Precision.HIGHEST


def _dotg(a, b):
  return lax.dot_general(a, b, (((0,), (0,)), ((), ())), precision=_HI)


def _dot(a, b):
  return jnp.dot(a, b, precision=_HI)


def _dotb(a, b):
  return lax.dot_general(a.astype(jnp.bfloat16), b.astype(jnp.bfloat16),
                         (((1,), (0,)), ((), ())),
                         preferred_element_type=jnp.float32)


def _silu(x):
  return x * (1.0 / (1.0 + jnp.exp(-x)))


def _coscut(r):
  return 0.5 * (jnp.cos(math.pi * r / _CUT) + 1.0) * (r < _CUT).astype(r.dtype)


def _fiota(shape, dim):
  return lax.broadcasted_iota(jnp.int32, shape, dim).astype(jnp.float32)


def _body(aux_ref, h_ref, zemb_ref, ef_ref,
          pw0_ref, pw0e2_ref, pb0b_ref, pw1b_ref, pb1b_ref, pw2b_ref,
          pb2b_ref,
          gw0_ref, gb0_ref, gw1_ref, gb1_ref, gw2_ref, gb2_ref,
          ow0_ref, ob0_ref, ow1_ref, ob1_ref,
          out_ref):
  f32 = jnp.float32
  aux = aux_ref[0]
  hblk = h_ref[0]

  lane = _fiota((_N, _N), 1)
  sub = _fiota((_N, _N), 0)
  eye = (sub == lane).astype(f32)
  sub_col = _fiota((_N, 1), 0)

  a_col = jnp.sum(aux * (lane == 5.0).astype(f32), axis=1, keepdims=True)
  m_col = jnp.sum(aux * (lane == 3.0).astype(f32), axis=1, keepdims=True)

  cmask3 = (lane < 3.0).astype(f32)
  p3 = aux * cmask3
  a_oh = (sub_col == a_col).astype(f32)
  pos0row = _dotg(a_oh, p3)

  d0 = p3 - pos0row
  r2 = jnp.sum(d0 * d0, axis=1, keepdims=True)
  r_col = jnp.sqrt(r2)
  r_row = _dotg(r_col, eye)

  valid_col = ((m_col > 0.5) & (sub_col != a_col) &
               (r_col <= _CUT)).astype(f32)
  valid_row = _dotg(valid_col, eye)

  d2 = jnp.zeros((_N, _N), f32)
  for d in range(3):
    x_col = jnp.sum(aux * (lane == float(d)).astype(f32), axis=1,
                    keepdims=True)
    x_row = _dotg(x_col, eye)
    dd = x_col - x_row
    d2 = d2 + dd * dd
  dmat = jnp.sqrt(d2)

  score = r_col + r_row + 0.5 * dmat
  tri = (sub < lane).astype(f32)
  pvb = (valid_col > 0.5) & (valid_row > 0.5) & (sub < lane)
  si = jnp.where(pvb, lax.bitcast_convert_type(score, jnp.int32), _INF_BITS)

  def bis(_, lohi):
    lo, hi = lohi
    d = (hi - lo) // 4
    m1 = lo + d
    m2 = lo + 2 * d
    m3 = lo + 3 * d
    c1 = jnp.sum((si <= m1).astype(jnp.int32)) >= _PMAX
    c2 = jnp.sum((si <= m2).astype(jnp.int32)) >= _PMAX
    c3 = jnp.sum((si <= m3).astype(jnp.int32)) >= _PMAX
    hi_n = jnp.where(c1, m1, jnp.where(c2, m2, jnp.where(c3, m3, hi)))
    lo_n = jnp.where(c1, lo,
                     jnp.where(c2, m1 + 1, jnp.where(c3, m2 + 1, m3 + 1)))
    return (lo_n, hi_n)

  _, thr = lax.fori_loop(0, 16, bis, (jnp.int32(0), jnp.int32(_INF_BITS)))

  s01 = (pvb & (si <= thr)).astype(f32)
  rank = _dotb(s01, tri)
  rc_col = jnp.sum(s01, axis=1, keepdims=True)
  off_col = _dotg(tri, rc_col)
  end_col = off_col + rc_col
  off_row = _dotg(off_col, eye)
  end_row = _dotg(end_col, eye)

  s_io = _fiota((_PMAX, _N), 0)
  rowsel = ((off_row <= s_io) & (s_io < end_row)).astype(f32)
  s_colv = _fiota((_PMAX, 1), 0)
  within = s_colv - _dotb(rowsel, off_col)
  rank_rows = _dotb(rowsel, rank)
  sel_rows = _dotb(rowsel, s01)
  ksel = ((rank_rows == within) & (sel_rows > 0.5)).astype(f32)
  used = jnp.sum(rowsel, axis=1, keepdims=True)

  gj = _dot(rowsel, aux)
  gk = _dot(ksel, aux)
  hj = _dotb(rowsel, hblk)
  hk = _dotb(ksel, hblk)

  lane256 = _fiota((_PMAX, _N), 1)
  c3 = (lane256 < 3.0).astype(f32)
  vj = (gj - pos0row) * c3
  vk = (gk - pos0row) * c3
  vjk = (gk - gj) * c3
  r0j = jnp.sqrt(jnp.sum(vj * vj, axis=1, keepdims=True))
  r0k = jnp.sqrt(jnp.sum(vk * vk, axis=1, keepdims=True))
  rjk = jnp.sqrt(jnp.sum(vjk * vjk, axis=1, keepdims=True))
  uj = vj / jnp.maximum(r0j, 1e-8)
  uk = vk / jnp.maximum(r0k, 1e-8)
  cosang = jnp.clip(jnp.sum(uj * uk, axis=1, keepdims=True), -1.0, 1.0)

  offs = _fiota((_PMAX, _RBF), 1) * (_CUT / (_RBF - 1))
  coeff = -0.5 / (_CUT / (_RBF - 1)) ** 2

  def rbf(r):
    rr = jnp.minimum(r, _CUT)
    return jnp.exp(coeff * (rr - offs) ** 2)

  f0j = rbf(r0j)
  f0k = rbf(r0k)
  fjk = rbf(rjk)

  gw0 = gw0_ref[...]
  pre = (_dotb(hj, gw0[0:128]) + _dotb(hk, gw0[128:256])
         + _dotb(f0j, gw0[256:288]) + _dotb(f0k, gw0[288:320])
         + _dotb(fjk, gw0[320:352]) + cosang * gw0[352:353]
         + gb0_ref[...])
  g1 = _silu(pre)
  g2 = _silu(_dotb(g1, gw1_ref[...]) + gb1_ref[...])
  ggeom = _dotb(g2, gw2_ref[...]) + gb2_ref[...]

  pw0 = pw0_ref[...]
  zemb = zemb_ref[...]
  zwj = _dotb(zemb, pw0[0:32])
  zwk = _dotb(zemb, pw0[32:64])
  efbp = _dot(ef_ref[...], pw0e2_ref[...]) + pb0b_ref[...]

  zj = jnp.sum(gj * (lane256 == 4.0).astype(f32), axis=1, keepdims=True)
  zk = jnp.sum(gk * (lane256 == 4.0).astype(f32), axis=1, keepdims=True)
  zjoh = (lane256 == zj).astype(f32)
  zkoh = (lane256 == zk).astype(f32)
  ej = _dotb(zjoh, zwj)
  ek = _dotb(zkoh, zwk)

  ejk = ej + ek
  ejk2 = jnp.concatenate([ejk, ejk], axis=1)
  pre0 = (ejk2[:, None, :] + efbp[None, :, :]).reshape(
      _PMAX * _NE // 2, 2 * _PHID)
  x1 = _silu(pre0)
  x2 = _silu(_dotb(x1, pw1b_ref[...]) + pb1b_ref[...])
  gep = _dotb(x2, pw2b_ref[...]) + pb2b_ref[...]

  ccc = _coscut(jnp.concatenate([r0j, r0k, rjk], axis=1))
  cw = ccc[:, 0:1] * ccc[:, 1:2] * ccc[:, 2:3]
  w = cw * used
  amat = ggeom * w
  am2 = jnp.concatenate([amat, amat], axis=1)
  ge3 = gep.reshape(_PMAX, _NE // 2, 2 * _SDIM)
  aggp = jnp.sum(ge3 * am2[:, None, :], axis=0)
  n64 = lax.broadcasted_iota(jnp.int32, (_NE, _NE // 2), 0)
  nn2 = lax.broadcasted_iota(jnp.int32, (_NE, _NE // 2), 1) * 2
  r_e = (n64 == nn2).astype(f32)
  r_o = (n64 == nn2 + 1).astype(f32)
  agg = _dot(r_e, aggp[:, :_SDIM]) + _dot(r_o, aggp[:, _SDIM:])
  norm = jnp.maximum(jnp.sum(w), 1e-8)
  agg = agg / norm

  o1 = _silu(_dot(agg, ow0_ref[...]) + ob0_ref[...])
  out = _dot(o1, ow1_ref[...]) + ob1_ref[...]
  out_ref[...] = out[None]


@jax.jit
def kernel(h, z, pos, mask, e_feat, z_emb,
           pair_w0, pair_b0, pair_w1, pair_b1, pair_w2, pair_b2,
           geom_w0, geom_b0, geom_w1, geom_b1, geom_w2, geom_b2,
           out_w0, out_b0, out_w1, out_b1, absorber_index):
  f32 = jnp.float32
  b = h.shape[0]
  aux = jnp.zeros((b, _N, _N), f32)
  aux = aux.at[:, :, 0:3].set(pos.astype(f32))
  aux = aux.at[:, :, 3].set(mask.astype(f32))
  aux = aux.at[:, :, 4].set(z.astype(f32))
  aux = aux.at[:, :, 5].set(jnp.asarray(absorber_index, f32))

  zemb_pad = jnp.zeros((_N, _ZEMB), f32).at[:z_emb.shape[0]].set(z_emb)

  ef_pack = jnp.concatenate([e_feat[0::2], e_feat[1::2]], axis=1)
  w0e = pair_w0[2 * _ZEMB:]
  pw0e2 = (jnp.zeros((2 * _EDIM, 2 * _PHID), f32)
           .at[:_EDIM, :_PHID].set(w0e).at[_EDIM:, _PHID:].set(w0e))
  pb0b = jnp.concatenate([pair_b0, pair_b0]).reshape(1, -1)

  pw1b = (jnp.zeros((2 * _PHID, 2 * _PHID), f32)
          .at[:_PHID, :_PHID].set(pair_w1).at[_PHID:, _PHID:].set(pair_w1))
  pb1b = jnp.concatenate([pair_b1, pair_b1]).reshape(1, -1)
  pw2b = (jnp.zeros((2 * _PHID, 2 * _SDIM), f32)
          .at[:_PHID, :_SDIM].set(pair_w2).at[_PHID:, _SDIM:].set(pair_w2))
  pb2b = jnp.concatenate([pair_b2, pair_b2]).reshape(1, -1)

  def fullspec(x):
    r = x.ndim
    return pl.BlockSpec(x.shape, lambda i, _r=r: (0,) * _r)

  ins = (aux, h, zemb_pad, ef_pack,
         pair_w0, pw0e2, pb0b, pw1b, pb1b, pw2b, pb2b,
         geom_w0, geom_b0.reshape(1, -1), geom_w1, geom_b1.reshape(1, -1),
         geom_w2, geom_b2.reshape(1, -1),
         out_w0, out_b0.reshape(1, -1), out_w1, out_b1.reshape(1, -1))

  specs = [pl.BlockSpec((1, _N, _N), lambda i: (i, 0, 0)),
           pl.BlockSpec((1, _N, _ATOM), lambda i: (i, 0, 0))]
  specs += [fullspec(x) for x in ins[2:]]

  return pl.pallas_call(
      _body,
      grid=(b,),
      in_specs=specs,
      out_specs=pl.BlockSpec((1, _NE, _ODIM), lambda i: (i, 0, 0)),
      out_shape=jax.ShapeDtypeStruct((b, _NE, _ODIM), f32),
  )(*ins)


# speedup vs baseline: 5.0730x; 1.1620x over previous
import functools
import math

import jax
import jax.numpy as jnp
from jax import lax
from jax.experimental import pallas as pl

_B = 16
_N = 128
_ATOM = 128
_RBF = 32
_GHID = 128
_SDIM = 32
_ODIM = 64
_CUT = 6.0
_PMAX = 256
_ZEMB = 32
_EDIM = 32
_NE = 64
_PHID = 64
_INF_BITS = 0x7F800000

_HI = lax.Precision.HIGHEST


def _dotg(a, b):
  return lax.dot_general(a, b, (((0,), (0,)), ((), ())), precision=_HI)


def _dot(a, b):
  return jnp.dot(a, b, precision=_HI)


def _dotb(a, b):
  return lax.dot_general(a.astype(jnp.bfloat16), b.astype(jnp.bfloat16),
                         (((1,), (0,)), ((), ())),
                         preferred_element_type=jnp.float32)


def _silu(x):
  return x * (1.0 / (1.0 + jnp.exp(-x)))


def _coscut(r):
  return 0.5 * (jnp.cos(math.pi * r / _CUT) + 1.0) * (r < _CUT).astype(r.dtype)


def _fiota(shape, dim):
  return lax.broadcasted_iota(jnp.int32, shape, dim).astype(jnp.float32)


def _body(aux_ref, h_ref, zemb_ref, ef_ref,
          pw0_ref, pw0e2_ref, pb0b_ref, pw1b_ref, pb1b_ref, pw2b_ref,
          pb2b_ref,
          gw0_ref, gb0_ref, gw1_ref, gb1_ref, gw2_ref, gb2_ref,
          ow0_ref, ob0_ref, ow1_ref, ob1_ref,
          out_ref):
  f32 = jnp.float32
  lane = _fiota((_N, _N), 1)
  sub = _fiota((_N, _N), 0)
  eye = (sub == lane).astype(f32)
  sub_col = _fiota((_N, 1), 0)
  cmask3 = (lane < 3.0).astype(f32)
  tri = (sub < lane).astype(f32)

  def stage1(aux):
    a_col = jnp.sum(aux * (lane == 5.0).astype(f32), axis=1, keepdims=True)
    m_col = jnp.sum(aux * (lane == 3.0).astype(f32), axis=1, keepdims=True)
    p3 = aux * cmask3
    a_oh = (sub_col == a_col).astype(f32)
    pos0row = _dotg(a_oh, p3)

    d0 = p3 - pos0row
    r2 = jnp.sum(d0 * d0, axis=1, keepdims=True)
    r_col = jnp.sqrt(r2)
    r_row = _dotg(r_col, eye)
    valid_col = ((m_col > 0.5) & (sub_col != a_col) &
                 (r_col <= _CUT)).astype(f32)
    valid_row = _dotg(valid_col, eye)

    d2 = jnp.zeros((_N, _N), f32)
    for d in range(3):
      x_col = jnp.sum(aux * (lane == float(d)).astype(f32), axis=1,
                      keepdims=True)
      x_row = _dotg(x_col, eye)
      dd = x_col - x_row
      d2 = d2 + dd * dd
    dmat = jnp.sqrt(d2)

    score = r_col + r_row + 0.5 * dmat
    pvb = (valid_col > 0.5) & (valid_row > 0.5) & (sub < lane)
    si = jnp.where(pvb, lax.bitcast_convert_type(score, jnp.int32), _INF_BITS)
    return si, pvb, pos0row

  si_a, pvb_a, pos0_a = stage1(aux_ref[0])
  si_b, pvb_b, pos0_b = stage1(aux_ref[1])

  def step(si, lo, hi):
    d = (hi - lo) // 4
    m1 = lo + d
    m2 = lo + 2 * d
    m3 = lo + 3 * d
    c1 = jnp.sum((si <= m1).astype(jnp.int32)) >= _PMAX
    c2 = jnp.sum((si <= m2).astype(jnp.int32)) >= _PMAX
    c3 = jnp.sum((si <= m3).astype(jnp.int32)) >= _PMAX
    hi_n = jnp.where(c1, m1, jnp.where(c2, m2, jnp.where(c3, m3, hi)))
    lo_n = jnp.where(c1, lo,
                     jnp.where(c2, m1 + 1, jnp.where(c3, m2 + 1, m3 + 1)))
    return lo_n, hi_n

  def bis(_, c):
    lo_a, hi_a, lo_b, hi_b = c
    lo_a, hi_a = step(si_a, lo_a, hi_a)
    lo_b, hi_b = step(si_b, lo_b, hi_b)
    return (lo_a, hi_a, lo_b, hi_b)

  z32 = jnp.int32(0)
  inf32 = jnp.int32(_INF_BITS)
  _, thr_a, _, thr_b = lax.fori_loop(0, 16, bis, (z32, inf32, z32, inf32))

  pw0 = pw0_ref[...]
  zemb = zemb_ref[...]
  zwj = _dotb(zemb, pw0[0:32])
  zwk = _dotb(zemb, pw0[32:64])
  efbp = _dot(ef_ref[...], pw0e2_ref[...]) + pb0b_ref[...]
  gw0 = gw0_ref[...]
  n64 = lax.broadcasted_iota(jnp.int32, (_NE, _NE // 2), 0)
  nn2 = lax.broadcasted_iota(jnp.int32, (_NE, _NE // 2), 1) * 2
  r_e = (n64 == nn2).astype(f32)
  r_o = (n64 == nn2 + 1).astype(f32)
  s_io = _fiota((_PMAX, _N), 0)
  s_colv = _fiota((_PMAX, 1), 0)
  lane256 = _fiota((_PMAX, _N), 1)
  c3m = (lane256 < 3.0).astype(f32)
  offs = _fiota((_PMAX, _RBF), 1) * (_CUT / (_RBF - 1))
  coeff = -0.5 / (_CUT / (_RBF - 1)) ** 2

  def stage2(si, pvb, pos0row, thr, aux, hblk):
    s01 = (pvb & (si <= thr)).astype(f32)
    rank = _dotb(s01, tri)
    rc_col = jnp.sum(s01, axis=1, keepdims=True)
    off_col = _dotg(tri, rc_col)
    end_col = off_col + rc_col
    off_row = _dotg(off_col, eye)
    end_row = _dotg(end_col, eye)

    rowsel = ((off_row <= s_io) & (s_io < end_row)).astype(f32)
    within = s_colv - _dotb(rowsel, off_col)
    rank_rows = _dotb(rowsel, rank)
    sel_rows = _dotb(rowsel, s01)
    ksel = ((rank_rows == within) & (sel_rows > 0.5)).astype(f32)
    used = jnp.sum(rowsel, axis=1, keepdims=True)

    gj = _dot(rowsel, aux)
    gk = _dot(ksel, aux)
    hj = _dotb(rowsel, hblk)
    hk = _dotb(ksel, hblk)

    vj = (gj - pos0row) * c3m
    vk = (gk - pos0row) * c3m
    vjk = (gk - gj) * c3m
    r0j = jnp.sqrt(jnp.sum(vj * vj, axis=1, keepdims=True))
    r0k = jnp.sqrt(jnp.sum(vk * vk, axis=1, keepdims=True))
    rjk = jnp.sqrt(jnp.sum(vjk * vjk, axis=1, keepdims=True))
    uj = vj / jnp.maximum(r0j, 1e-8)
    uk = vk / jnp.maximum(r0k, 1e-8)
    cosang = jnp.clip(jnp.sum(uj * uk, axis=1, keepdims=True), -1.0, 1.0)

    def rbf(r):
      rr = jnp.minimum(r, _CUT)
      return jnp.exp(coeff * (rr - offs) ** 2)

    f0j = rbf(r0j)
    f0k = rbf(r0k)
    fjk = rbf(rjk)

    pre = (_dotb(hj, gw0[0:128]) + _dotb(hk, gw0[128:256])
           + _dotb(f0j, gw0[256:288]) + _dotb(f0k, gw0[288:320])
           + _dotb(fjk, gw0[320:352]) + cosang * gw0[352:353]
           + gb0_ref[...])
    g1 = _silu(pre)
    g2 = _silu(_dotb(g1, gw1_ref[...]) + gb1_ref[...])
    ggeom = _dotb(g2, gw2_ref[...]) + gb2_ref[...]

    zj = jnp.sum(gj * (lane256 == 4.0).astype(f32), axis=1, keepdims=True)
    zk = jnp.sum(gk * (lane256 == 4.0).astype(f32), axis=1, keepdims=True)
    zjoh = (lane256 == zj).astype(f32)
    zkoh = (lane256 == zk).astype(f32)
    ej = _dotb(zjoh, zwj)
    ek = _dotb(zkoh, zwk)

    ejk = ej + ek
    ejk2 = jnp.concatenate([ejk, ejk], axis=1)
    pre0 = (ejk2[:, None, :] + efbp[None, :, :]).reshape(
        _PMAX * _NE // 2, 2 * _PHID)
    x1 = _silu(pre0)
    x2 = _silu(_dotb(x1, pw1b_ref[...]) + pb1b_ref[...])
    gep = _dotb(x2, pw2b_ref[...]) + pb2b_ref[...]

    ccc = _coscut(jnp.concatenate([r0j, r0k, rjk], axis=1))
    cw = ccc[:, 0:1] * ccc[:, 1:2] * ccc[:, 2:3]
    w = cw * used
    amat = ggeom * w
    am2 = jnp.concatenate([amat, amat], axis=1)
    ge3 = gep.reshape(_PMAX, _NE // 2, 2 * _SDIM)
    aggp = jnp.sum(ge3 * am2[:, None, :], axis=0)
    agg = _dot(r_e, aggp[:, :_SDIM]) + _dot(r_o, aggp[:, _SDIM:])
    norm = jnp.maximum(jnp.sum(w), 1e-8)
    agg = agg / norm

    o1 = _silu(_dot(agg, ow0_ref[...]) + ob0_ref[...])
    return _dot(o1, ow1_ref[...]) + ob1_ref[...]

  out_a = stage2(si_a, pvb_a, pos0_a, thr_a, aux_ref[0], h_ref[0])
  out_b = stage2(si_b, pvb_b, pos0_b, thr_b, aux_ref[1], h_ref[1])
  out_ref[0:1] = out_a[None]
  out_ref[1:2] = out_b[None]


@jax.jit
def kernel(h, z, pos, mask, e_feat, z_emb,
           pair_w0, pair_b0, pair_w1, pair_b1, pair_w2, pair_b2,
           geom_w0, geom_b0, geom_w1, geom_b1, geom_w2, geom_b2,
           out_w0, out_b0, out_w1, out_b1, absorber_index):
  f32 = jnp.float32
  b = h.shape[0]
  aux = jnp.zeros((b, _N, _N), f32)
  aux = aux.at[:, :, 0:3].set(pos.astype(f32))
  aux = aux.at[:, :, 3].set(mask.astype(f32))
  aux = aux.at[:, :, 4].set(z.astype(f32))
  aux = aux.at[:, :, 5].set(jnp.asarray(absorber_index, f32))

  zemb_pad = jnp.zeros((_N, _ZEMB), f32).at[:z_emb.shape[0]].set(z_emb)

  ef_pack = jnp.concatenate([e_feat[0::2], e_feat[1::2]], axis=1)
  w0e = pair_w0[2 * _ZEMB:]
  pw0e2 = (jnp.zeros((2 * _EDIM, 2 * _PHID), f32)
           .at[:_EDIM, :_PHID].set(w0e).at[_EDIM:, _PHID:].set(w0e))
  pb0b = jnp.concatenate([pair_b0, pair_b0]).reshape(1, -1)

  pw1b = (jnp.zeros((2 * _PHID, 2 * _PHID), f32)
          .at[:_PHID, :_PHID].set(pair_w1).at[_PHID:, _PHID:].set(pair_w1))
  pb1b = jnp.concatenate([pair_b1, pair_b1]).reshape(1, -1)
  pw2b = (jnp.zeros((2 * _PHID, 2 * _SDIM), f32)
          .at[:_PHID, :_SDIM].set(pair_w2).at[_PHID:, _SDIM:].set(pair_w2))
  pb2b = jnp.concatenate([pair_b2, pair_b2]).reshape(1, -1)

  def fullspec(x):
    r = x.ndim
    return pl.BlockSpec(x.shape, lambda i, _r=r: (0,) * _r)

  ins = (aux, h, zemb_pad, ef_pack,
         pair_w0, pw0e2, pb0b, pw1b, pb1b, pw2b, pb2b,
         geom_w0, geom_b0.reshape(1, -1), geom_w1, geom_b1.reshape(1, -1),
         geom_w2, geom_b2.reshape(1, -1),
         out_w0, out_b0.reshape(1, -1), out_w1, out_b1.reshape(1, -1))

  specs = [pl.BlockSpec((2, _N, _N), lambda i: (i, 0, 0)),
           pl.BlockSpec((2, _N, _ATOM), lambda i: (i, 0, 0))]
  specs += [fullspec(x) for x in ins[2:]]

  return pl.pallas_call(
      _body,
      grid=(b // 2,),
      in_specs=specs,
      out_specs=pl.BlockSpec((2, _NE, _ODIM), lambda i: (i, 0, 0)),
      out_shape=jax.ShapeDtypeStruct((b, _NE, _ODIM), f32),
  )(*ins)


# speedup vs baseline: 5.4654x; 1.0774x over previous
import functools
import math

import jax
import jax.numpy as jnp
from jax import lax
from jax.experimental import pallas as pl

_B = 16
_N = 128
_ATOM = 128
_RBF = 32
_GHID = 128
_SDIM = 32
_ODIM = 64
_CUT = 6.0
_PMAX = 256
_ZEMB = 32
_EDIM = 32
_NE = 64
_PHID = 64
_INF_BITS = 0x7F800000
_TPB = 4

_HI = lax.Precision.HIGHEST


def _dotg(a, b):
  return lax.dot_general(a, b, (((0,), (0,)), ((), ())), precision=_HI)


def _dot(a, b):
  return jnp.dot(a, b, precision=_HI)


def _dotb(a, b):
  return lax.dot_general(a.astype(jnp.bfloat16), b.astype(jnp.bfloat16),
                         (((1,), (0,)), ((), ())),
                         preferred_element_type=jnp.float32)


def _silu(x):
  return x * (1.0 / (1.0 + jnp.exp(-x)))


def _coscut(r):
  return 0.5 * (jnp.cos(math.pi * r / _CUT) + 1.0) * (r < _CUT).astype(r.dtype)


def _fiota(shape, dim):
  return lax.broadcasted_iota(jnp.int32, shape, dim).astype(jnp.float32)


def _body(aux_ref, h_ref, zemb_ref, ef_ref,
          pw0_ref, pw0e2_ref, pb0b_ref, pw1b_ref, pb1b_ref, pw2b_ref,
          pb2b_ref,
          gw0_ref, gb0_ref, gw1_ref, gb1_ref, gw2_ref, gb2_ref,
          ow0_ref, ob0_ref, ow1_ref, ob1_ref,
          out_ref):
  f32 = jnp.float32
  lane = _fiota((_N, _N), 1)
  sub = _fiota((_N, _N), 0)
  eye = (sub == lane).astype(f32)
  sub_col = _fiota((_N, 1), 0)
  cmask3 = (lane < 3.0).astype(f32)
  tri = (sub < lane).astype(f32)

  def stage1(aux):
    a_col = jnp.sum(aux * (lane == 5.0).astype(f32), axis=1, keepdims=True)
    m_col = jnp.sum(aux * (lane == 3.0).astype(f32), axis=1, keepdims=True)
    p3 = aux * cmask3
    a_oh = (sub_col == a_col).astype(f32)
    pos0row = _dotg(a_oh, p3)

    d0 = p3 - pos0row
    r2 = jnp.sum(d0 * d0, axis=1, keepdims=True)
    r_col = jnp.sqrt(r2)
    r_row = _dotg(r_col, eye)
    valid_col = ((m_col > 0.5) & (sub_col != a_col) &
                 (r_col <= _CUT)).astype(f32)
    valid_row = _dotg(valid_col, eye)

    d2 = jnp.zeros((_N, _N), f32)
    for d in range(3):
      x_col = jnp.sum(aux * (lane == float(d)).astype(f32), axis=1,
                      keepdims=True)
      x_row = _dotg(x_col, eye)
      dd = x_col - x_row
      d2 = d2 + dd * dd
    dmat = jnp.sqrt(d2)

    score = r_col + r_row + 0.5 * dmat
    pvb = (valid_col > 0.5) & (valid_row > 0.5) & (sub < lane)
    si = jnp.where(pvb, lax.bitcast_convert_type(score, jnp.int32), _INF_BITS)
    return si, pvb, pos0row

  s1 = [stage1(aux_ref[t]) for t in range(_TPB)]

  def step(si, lo, hi):
    d = (hi - lo) // 4
    m1 = lo + d
    m2 = lo + 2 * d
    m3 = lo + 3 * d
    c1 = jnp.sum((si <= m1).astype(jnp.int32)) >= _PMAX
    c2 = jnp.sum((si <= m2).astype(jnp.int32)) >= _PMAX
    c3 = jnp.sum((si <= m3).astype(jnp.int32)) >= _PMAX
    hi_n = jnp.where(c1, m1, jnp.where(c2, m2, jnp.where(c3, m3, hi)))
    lo_n = jnp.where(c1, lo,
                     jnp.where(c2, m1 + 1, jnp.where(c3, m2 + 1, m3 + 1)))
    return lo_n, hi_n

  def bis(_, c):
    out = []
    for t in range(_TPB):
      lo, hi = step(s1[t][0], c[2 * t], c[2 * t + 1])
      out += [lo, hi]
    return tuple(out)

  z32 = jnp.int32(0)
  inf32 = jnp.int32(_INF_BITS)
  fin = lax.fori_loop(0, 16, bis, (z32, inf32) * _TPB)
  thrs = [fin[2 * t + 1] for t in range(_TPB)]

  pw0 = pw0_ref[...]
  zemb = zemb_ref[...]
  zwj = _dotb(zemb, pw0[0:32])
  zwk = _dotb(zemb, pw0[32:64])
  efbp = _dot(ef_ref[...], pw0e2_ref[...]) + pb0b_ref[...]
  gw0 = gw0_ref[...]
  n64 = lax.broadcasted_iota(jnp.int32, (_NE, _NE // 2), 0)
  nn2 = lax.broadcasted_iota(jnp.int32, (_NE, _NE // 2), 1) * 2
  r_e = (n64 == nn2).astype(f32)
  r_o = (n64 == nn2 + 1).astype(f32)
  s_io = _fiota((_PMAX, _N), 0)
  s_colv = _fiota((_PMAX, 1), 0)
  lane256 = _fiota((_PMAX, _N), 1)
  c3m = (lane256 < 3.0).astype(f32)
  offs = _fiota((_PMAX, _RBF), 1) * (_CUT / (_RBF - 1))
  coeff = -0.5 / (_CUT / (_RBF - 1)) ** 2

  def stage2(si, pvb, pos0row, thr, aux, hblk):
    s01 = (pvb & (si <= thr)).astype(f32)
    rank = _dotb(s01, tri)
    rc_col = jnp.sum(s01, axis=1, keepdims=True)
    off_col = _dotg(tri, rc_col)
    end_col = off_col + rc_col
    off_row = _dotg(off_col, eye)
    end_row = _dotg(end_col, eye)

    rowsel = ((off_row <= s_io) & (s_io < end_row)).astype(f32)
    within = s_colv - _dotb(rowsel, off_col)
    rank_rows = _dotb(rowsel, rank)
    sel_rows = _dotb(rowsel, s01)
    ksel = ((rank_rows == within) & (sel_rows > 0.5)).astype(f32)
    used = jnp.sum(rowsel, axis=1, keepdims=True)

    gj = _dot(rowsel, aux)
    gk = _dot(ksel, aux)
    hj = _dotb(rowsel, hblk)
    hk = _dotb(ksel, hblk)

    vj = (gj - pos0row) * c3m
    vk = (gk - pos0row) * c3m
    vjk = (gk - gj) * c3m
    r0j = jnp.sqrt(jnp.sum(vj * vj, axis=1, keepdims=True))
    r0k = jnp.sqrt(jnp.sum(vk * vk, axis=1, keepdims=True))
    rjk = jnp.sqrt(jnp.sum(vjk * vjk, axis=1, keepdims=True))
    uj = vj / jnp.maximum(r0j, 1e-8)
    uk = vk / jnp.maximum(r0k, 1e-8)
    cosang = jnp.clip(jnp.sum(uj * uk, axis=1, keepdims=True), -1.0, 1.0)

    def rbf(r):
      rr = jnp.minimum(r, _CUT)
      return jnp.exp(coeff * (rr - offs) ** 2)

    f0j = rbf(r0j)
    f0k = rbf(r0k)
    fjk = rbf(rjk)

    pre = (_dotb(hj, gw0[0:128]) + _dotb(hk, gw0[128:256])
           + _dotb(f0j, gw0[256:288]) + _dotb(f0k, gw0[288:320])
           + _dotb(fjk, gw0[320:352]) + cosang * gw0[352:353]
           + gb0_ref[...])
    g1 = _silu(pre)
    g2 = _silu(_dotb(g1, gw1_ref[...]) + gb1_ref[...])
    ggeom = _dotb(g2, gw2_ref[...]) + gb2_ref[...]

    zj = jnp.sum(gj * (lane256 == 4.0).astype(f32), axis=1, keepdims=True)
    zk = jnp.sum(gk * (lane256 == 4.0).astype(f32), axis=1, keepdims=True)
    zjoh = (lane256 == zj).astype(f32)
    zkoh = (lane256 == zk).astype(f32)
    ej = _dotb(zjoh, zwj)
    ek = _dotb(zkoh, zwk)

    ejk = ej + ek
    ejk2 = jnp.concatenate([ejk, ejk], axis=1)
    pre0 = (ejk2[:, None, :] + efbp[None, :, :]).reshape(
        _PMAX * _NE // 2, 2 * _PHID)
    x1 = _silu(pre0)
    x2 = _silu(_dotb(x1, pw1b_ref[...]) + pb1b_ref[...])
    gep = _dotb(x2, pw2b_ref[...]) + pb2b_ref[...]

    ccc = _coscut(jnp.concatenate([r0j, r0k, rjk], axis=1))
    cw = ccc[:, 0:1] * ccc[:, 1:2] * ccc[:, 2:3]
    w = cw * used
    amat = ggeom * w
    am2 = jnp.concatenate([amat, amat], axis=1)
    ge3 = gep.reshape(_PMAX, _NE // 2, 2 * _SDIM)
    aggp = jnp.sum(ge3 * am2[:, None, :], axis=0)
    agg = _dot(r_e, aggp[:, :_SDIM]) + _dot(r_o, aggp[:, _SDIM:])
    norm = jnp.maximum(jnp.sum(w), 1e-8)
    agg = agg / norm

    o1 = _silu(_dot(agg, ow0_ref[...]) + ob0_ref[...])
    return _dot(o1, ow1_ref[...]) + ob1_ref[...]

  for t in range(_TPB):
    si_t, pvb_t, pos0_t = s1[t]
    out_t = stage2(si_t, pvb_t, pos0_t, thrs[t], aux_ref[t], h_ref[t])
    out_ref[t:t + 1] = out_t[None]


@jax.jit
def kernel(h, z, pos, mask, e_feat, z_emb,
           pair_w0, pair_b0, pair_w1, pair_b1, pair_w2, pair_b2,
           geom_w0, geom_b0, geom_w1, geom_b1, geom_w2, geom_b2,
           out_w0, out_b0, out_w1, out_b1, absorber_index):
  f32 = jnp.float32
  b = h.shape[0]
  aux = jnp.zeros((b, _N, _N), f32)
  aux = aux.at[:, :, 0:3].set(pos.astype(f32))
  aux = aux.at[:, :, 3].set(mask.astype(f32))
  aux = aux.at[:, :, 4].set(z.astype(f32))
  aux = aux.at[:, :, 5].set(jnp.asarray(absorber_index, f32))

  zemb_pad = jnp.zeros((_N, _ZEMB), f32).at[:z_emb.shape[0]].set(z_emb)

  ef_pack = jnp.concatenate([e_feat[0::2], e_feat[1::2]], axis=1)
  w0e = pair_w0[2 * _ZEMB:]
  pw0e2 = (jnp.zeros((2 * _EDIM, 2 * _PHID), f32)
           .at[:_EDIM, :_PHID].set(w0e).at[_EDIM:, _PHID:].set(w0e))
  pb0b = jnp.concatenate([pair_b0, pair_b0]).reshape(1, -1)

  pw1b = (jnp.zeros((2 * _PHID, 2 * _PHID), f32)
          .at[:_PHID, :_PHID].set(pair_w1).at[_PHID:, _PHID:].set(pair_w1))
  pb1b = jnp.concatenate([pair_b1, pair_b1]).reshape(1, -1)
  pw2b = (jnp.zeros((2 * _PHID, 2 * _SDIM), f32)
          .at[:_PHID, :_SDIM].set(pair_w2).at[_PHID:, _SDIM:].set(pair_w2))
  pb2b = jnp.concatenate([pair_b2, pair_b2]).reshape(1, -1)

  def fullspec(x):
    r = x.ndim
    return pl.BlockSpec(x.shape, lambda i, _r=r: (0,) * _r)

  ins = (aux, h, zemb_pad, ef_pack,
         pair_w0, pw0e2, pb0b, pw1b, pb1b, pw2b, pb2b,
         geom_w0, geom_b0.reshape(1, -1), geom_w1, geom_b1.reshape(1, -1),
         geom_w2, geom_b2.reshape(1, -1),
         out_w0, out_b0.reshape(1, -1), out_w1, out_b1.reshape(1, -1))

  specs = [pl.BlockSpec((_TPB, _N, _N), lambda i: (i, 0, 0)),
           pl.BlockSpec((_TPB, _N, _ATOM), lambda i: (i, 0, 0))]
  specs += [fullspec(x) for x in ins[2:]]

  return pl.pallas_call(
      _body,
      grid=(b // _TPB,),
      in_specs=specs,
      out_specs=pl.BlockSpec((_TPB, _NE, _ODIM), lambda i: (i, 0, 0)),
      out_shape=jax.ShapeDtypeStruct((b, _NE, _ODIM), f32),
  )(*ins)
